# baseline (speedup 1.0000x reference)
import functools

import jax
import jax.numpy as jnp
from jax import lax
from jax.experimental import pallas as pl
from jax.experimental.pallas import tpu as pltpu
from jax.experimental.pallas import tpu_sc as plsc

VOCAB = 1000
EMBED = 64
BATCH = 16384
HIST = 50
HIST_PAD = 56
LANE_PAD = 128

_INFO = plsc.get_sparse_core_info()
_NC = _INFO.num_cores
_NS = _INFO.num_subcores
_NW = _NC * _NS

_ROWS = BATCH * HIST_PAD
_BATCH_PER_W = BATCH // _NW
_NB = 4
_NCHUNK = _BATCH_PER_W // _NB
_PCHUNK = _NB * HIST_PAD


def _make_kernel():
  mesh = plsc.VectorSubcoreMesh(core_axis_name="c", subcore_axis_name="s")

  @functools.partial(
      pl.kernel,
      mesh=mesh,
      compiler_params=pltpu.CompilerParams(use_tc_tiling_on_sc=False),
      out_type=jax.ShapeDtypeStruct((_ROWS, LANE_PAD), jnp.float32),
      scratch_types=[
          pltpu.VMEM((_BATCH_PER_W, HIST), jnp.int32),
          pltpu.VMEM((2, _NB, HIST, EMBED), jnp.float32),
          pltpu.VMEM((2, _PCHUNK, LANE_PAD), jnp.float32),
          pltpu.SemaphoreType.DMA,
          pltpu.SemaphoreType.DMA,
          pltpu.SemaphoreType.DMA,
      ],
  )
  def gather_kernel(idx_hbm, table_hbm, out_hbm, idx_all, abuf, bbuf,
                    gsem, s0, s1):
    wid = lax.axis_index("s") * _NC + lax.axis_index("c")
    base = wid * _BATCH_PER_W * HIST_PAD
    ssems = (s0, s1)

    def fire_gather(c, b):
      return [
          pltpu.async_copy(
              table_hbm.at[idx_all.at[c * _NB + j]],
              abuf.at[b].at[j],
              gsem,
          )
          for j in range(_NB)
      ]

    def repack(b):
      for j in range(_NB):

        def rbody(i, carry, j=j):
          h0 = i * 10
          for k in range(10):
            h = h0 + k
            br = j * HIST_PAD + h
            for l in range(0, EMBED, 16):
              bbuf[b, br, pl.ds(l, 16)] = abuf[b, j, h, pl.ds(l, 16)]
          return carry

        lax.fori_loop(0, HIST // 10, rbody, 0)

    def fire_store(c, b):
      pltpu.async_copy(
          bbuf.at[b], out_hbm.at[pl.ds(base + c * _PCHUNK, _PCHUNK)], ssems[b]
      )

    def wait_store(b):
      pltpu.make_async_copy(
          bbuf.at[b], out_hbm.at[pl.ds(0, _PCHUNK)], ssems[b]
      ).wait()

    pltpu.sync_copy(idx_hbm.at[wid], idx_all)
    for cp in fire_gather(0, 0):
      cp.wait()

    def step(c, b, o, fire_next):
      nxt = fire_gather(c + 1, o) if fire_next else []

      if isinstance(c, int):
        if c >= 2:
          wait_store(b)
      else:

        @pl.when(c >= 2)
        def _():
          wait_store(b)

      repack(b)
      fire_store(c, b)
      for cp in nxt:
        cp.wait()

    def pair_body(p, carry):
      for b in range(2):
        c = 2 * p + b
        step(c, b, 1 - b, True)
      return carry

    lax.fori_loop(0, (_NCHUNK - 2) // 2, pair_body, 0)
    step(_NCHUNK - 2, 0, 1, True)
    step(_NCHUNK - 1, 1, 0, False)
    wait_store(0)
    wait_store(1)

  return gather_kernel


_GATHER = _make_kernel()


def kernel(word_idx_list, W):
  idx = word_idx_list.astype(jnp.int32).reshape(_NW, _BATCH_PER_W, HIST)
  out = _GATHER(idx, W)
  return out.reshape(BATCH, HIST_PAD, LANE_PAD)[:, :HIST, :EMBED]


# speedup vs baseline: 1.0230x; 1.0230x over previous
import functools

import jax
import jax.numpy as jnp
from jax import lax
from jax.experimental import pallas as pl
from jax.experimental.pallas import tpu as pltpu
from jax.experimental.pallas import tpu_sc as plsc

VOCAB = 1000
EMBED = 64
BATCH = 16384
HIST = 50
HIST_PAD = 56
LANE_PAD = 128

_INFO = plsc.get_sparse_core_info()
_NC = _INFO.num_cores
_NS = _INFO.num_subcores
_NW = _NC * _NS

_ROWS = BATCH * HIST_PAD
_BATCH_PER_W = BATCH // _NW
_NB = 4
_NCHUNK = _BATCH_PER_W // _NB
_PCHUNK = _NB * HIST_PAD


def _make_kernel():
  mesh = plsc.VectorSubcoreMesh(core_axis_name="c", subcore_axis_name="s")

  @functools.partial(
      pl.kernel,
      mesh=mesh,
      compiler_params=pltpu.CompilerParams(use_tc_tiling_on_sc=False),
      out_type=jax.ShapeDtypeStruct((_ROWS, LANE_PAD), jnp.float32),
      scratch_types=[
          pltpu.VMEM((_BATCH_PER_W, HIST), jnp.int32),
          pltpu.VMEM((2, _NB, HIST, EMBED), jnp.float32),
          pltpu.VMEM((2, _PCHUNK, LANE_PAD), jnp.float32),
          pltpu.SemaphoreType.DMA,
          pltpu.SemaphoreType.DMA,
          pltpu.SemaphoreType.DMA,
      ],
  )
  def gather_kernel(idx_hbm, table_hbm, out_hbm, idx_all, abuf, bbuf,
                    gsem, s0, s1):
    wid = lax.axis_index("s") * _NC + lax.axis_index("c")
    base = wid * _BATCH_PER_W * HIST_PAD
    ssems = (s0, s1)

    def fire_gather(c, b):
      return [
          pltpu.async_copy(
              table_hbm.at[idx_all.at[c * _NB + j]],
              abuf.at[b].at[j],
              gsem,
          )
          for j in range(_NB)
      ]

    def repack(b):
      for j in range(_NB):

        def rbody(i, carry, j=j):
          h0 = i * 10
          for k in range(10):
            h = h0 + k
            br = j * HIST_PAD + h
            for l in range(0, EMBED, 16):
              bbuf[b, br, pl.ds(l, 16)] = abuf[b, j, h, pl.ds(l, 16)]
          return carry

        lax.fori_loop(0, HIST // 10, rbody, 0)

    def fire_store(c, b):
      for j in range(_NB):
        pltpu.async_copy(
            bbuf.at[b].at[pl.ds(j * HIST_PAD, HIST)],
            out_hbm.at[pl.ds(base + c * _PCHUNK + j * HIST_PAD, HIST)],
            ssems[b],
        )

    def wait_store(b):
      for j in range(_NB):
        pltpu.make_async_copy(
            bbuf.at[b].at[pl.ds(j * HIST_PAD, HIST)],
            out_hbm.at[pl.ds(0, HIST)],
            ssems[b],
        ).wait()

    pltpu.sync_copy(idx_hbm.at[wid], idx_all)
    for cp in fire_gather(0, 0):
      cp.wait()

    def step(c, b, o, fire_next):
      nxt = fire_gather(c + 1, o) if fire_next else []

      if isinstance(c, int):
        if c >= 2:
          wait_store(b)
      else:

        @pl.when(c >= 2)
        def _():
          wait_store(b)

      repack(b)
      fire_store(c, b)
      for cp in nxt:
        cp.wait()

    def pair_body(p, carry):
      for b in range(2):
        c = 2 * p + b
        step(c, b, 1 - b, True)
      return carry

    lax.fori_loop(0, (_NCHUNK - 2) // 2, pair_body, 0)
    step(_NCHUNK - 2, 0, 1, True)
    step(_NCHUNK - 1, 1, 0, False)
    wait_store(0)
    wait_store(1)

  return gather_kernel


_GATHER = _make_kernel()


def kernel(word_idx_list, W):
  idx = word_idx_list.astype(jnp.int32).reshape(_NW, _BATCH_PER_W, HIST)
  out = _GATHER(idx, W)
  return out.reshape(BATCH, HIST_PAD, LANE_PAD)[:, :HIST, :EMBED]
